# baseline (speedup 1.0000x reference)
import functools

import jax
import jax.numpy as jnp
from jax import lax
from jax.experimental import pallas as pl
from jax.experimental.pallas import tpu as pltpu
from jax.experimental.pallas import tpu_sc as plsc

F32 = jnp.float32


def _pad_hw(x, lo, hi):
    return jnp.pad(x, ((0, 0), (lo, hi), (lo, hi), (0, 0)))


def _s2d(x):
    B, H, W, C = x.shape
    x = x.reshape(B, H // 2, 2, W // 2, 2, C)
    x = x.transpose(0, 1, 3, 2, 4, 5)
    return x.reshape(B, H // 2, W // 2, 4 * C)


def _flatten_rows(x, extra):
    B, H, W, C = x.shape
    x = x.reshape(B, H * W, C)
    return jnp.pad(x, ((0, 0), (0, extra), (0, 0)))


def _conv_w(w, s2d):
    O, I, kh, kw = w.shape
    wt = w.transpose(2, 3, 1, 0)
    if s2d:
        wt = wt.reshape(2, 2, 2, 2, I, O)
        wt = wt.transpose(0, 2, 1, 3, 4, 5)
        return wt.reshape(kh * kw * I, O)
    return wt.reshape(kh * kw * I, O)


def _deconv_phase_w(dw):
    kidx = ((3, 1), (2, 0))
    phases = []
    for ry in (0, 1):
        for rx in (0, 1):
            blocks = [dw[:, :, kidx[ry][dy], kidx[rx][dx]]
                      for dy in (0, 1) for dx in (0, 1)]
            phases.append(jnp.concatenate(blocks, axis=0))
    return jnp.stack(phases)


def _tapconv_body(x_ref, w_ref, b_ref, o_ref, *, offs, M, Ho, Wp, Wo, act):
    xc = jnp.concatenate([x_ref[0, pl.ds(o, M), :] for o in offs], axis=1)
    acc = jnp.dot(xc, w_ref[...], preferred_element_type=F32)
    acc = acc + b_ref[...]
    if act == "relu":
        acc = jnp.maximum(acc, 0.0)
    C = acc.shape[-1]
    o_ref[0] = acc.reshape(Ho, Wp, C)[:, :Wo, :]


def _tapconv(x_flat, w, b, offs, M, Ho, Wp, Wo, act):
    B, N, Cin = x_flat.shape
    Cout = w.shape[-1]
    body = functools.partial(_tapconv_body, offs=offs, M=M, Ho=Ho, Wp=Wp,
                             Wo=Wo, act=act)
    return pl.pallas_call(
        body,
        grid=(B,),
        in_specs=[
            pl.BlockSpec((1, N, Cin), lambda i: (i, 0, 0)),
            pl.BlockSpec(w.shape, lambda i: (0, 0)),
            pl.BlockSpec(b.shape, lambda i: (0, 0)),
        ],
        out_specs=pl.BlockSpec((1, Ho, Wo, Cout), lambda i: (i, 0, 0, 0)),
        out_shape=jax.ShapeDtypeStruct((B, Ho, Wo, Cout), F32),
    )(x_flat, w, b)


def _deconv_body(x_ref, w_ref, b_ref, o_ref, *, offs4, M, Ho, Wp, Wo, act):
    accs = []
    for ph in range(4):
        xc = jnp.concatenate([x_ref[0, pl.ds(o, M), :] for o in offs4[ph]],
                             axis=1)
        acc = jnp.dot(xc, w_ref[ph], preferred_element_type=F32)
        acc = acc + b_ref[...]
        if act == "relu":
            acc = jnp.maximum(acc, 0.0)
        accs.append(acc)
    cat = jnp.concatenate(accs, axis=1)
    C4 = cat.shape[-1]
    o_ref[0] = cat.reshape(Ho, Wp, C4)[:, :Wo, :]


def _deconv(x_flat, wph, b, offs4, M, Ho, Wp, Wo, act):
    B, N, Cin = x_flat.shape
    Cout = wph.shape[-1]
    body = functools.partial(_deconv_body, offs4=offs4, M=M, Ho=Ho, Wp=Wp,
                             Wo=Wo, act=act)
    return pl.pallas_call(
        body,
        grid=(B,),
        in_specs=[
            pl.BlockSpec((1, N, Cin), lambda i: (i, 0, 0)),
            pl.BlockSpec(wph.shape, lambda i: (0, 0, 0)),
            pl.BlockSpec(b.shape, lambda i: (0, 0)),
        ],
        out_specs=pl.BlockSpec((1, Ho, Wo, 4 * Cout), lambda i: (i, 0, 0, 0)),
        out_shape=jax.ShapeDtypeStruct((B, Ho, Wo, 4 * Cout), F32),
    )(x_flat, wph, b)


def _d2s(x, C):
    B, Mh, Nw, _ = x.shape
    x = x.reshape(B, Mh, Nw, 2, 2, C).transpose(0, 1, 3, 2, 4, 5)
    return x.reshape(B, 2 * Mh, 2 * Nw, C)


def _dec12_body(q_ref, w1_ref, b1_ref, w2_ref, b2_ref, w3_ref, b3_ref,
                o_ref, s_ref, *, offs1, offs2):
    M1 = 56 * 58
    accs = []
    for ph in range(4):
        xc = jnp.concatenate([q_ref[0, pl.ds(o, M1), :] for o in offs1[ph]],
                             axis=1)
        acc = jnp.dot(xc, w1_ref[ph], preferred_element_type=F32)
        accs.append(jnp.maximum(acc + b1_ref[...], 0.0))
    cat = jnp.concatenate(accs, axis=1)
    v = cat.reshape(56, 58, 2, 2, 64).transpose(0, 2, 1, 3, 4)
    v = v.reshape(112, 116, 64)
    v = jnp.concatenate([jnp.zeros((112, 1, 64), F32), v[:, :112, :],
                         jnp.zeros((112, 1, 64), F32)],
                        axis=1)
    zrow = jnp.zeros((1, 114, 64), F32)
    v = jnp.concatenate([zrow, v, zrow], axis=0)
    vf = jnp.concatenate([v.reshape(12996, 64), jnp.zeros((4, 64), F32)],
                         axis=0)

    M2 = 112 * 114
    accs2 = []
    for ph in range(4):
        xc = jnp.concatenate([vf[o:o + M2, :] for o in offs2[ph]], axis=1)
        acc = jnp.dot(xc, w2_ref[ph], preferred_element_type=F32)
        accs2.append(jnp.maximum(acc + b2_ref[...], 0.0))
    cat2 = jnp.concatenate(accs2, axis=1)
    col = lax.rem(lax.broadcasted_iota(jnp.int32, (M2, 1), 0),
                  jnp.int32(114))
    cat2 = jnp.where(col < 112, cat2, 0.0)
    s_ref[0:115, :] = jnp.zeros((115, 128), F32)
    s_ref[pl.ds(115, M2), :] = cat2
    s_ref[12883:13000, :] = jnp.zeros((117, 128), F32)

    acc = jnp.zeros((M2, 12), F32) + b3_ref[...]
    for tap in range(9):
        dm, dn = tap // 3, tap % 3
        acc = acc + jnp.dot(s_ref[pl.ds(dm * 114 + dn, M2), :], w3_ref[tap],
                            preferred_element_type=F32)
    acc = jnp.tanh(acc)
    o_ref[0] = acc.reshape(112, 114, 12)[:, :112, :]


def _dec12(q_flat, w1, b1, w2, b2, w3, b3, offs1, offs2):
    B, N, _ = q_flat.shape
    body = functools.partial(_dec12_body, offs1=offs1, offs2=offs2)
    return pl.pallas_call(
        body,
        grid=(B,),
        in_specs=[
            pl.BlockSpec((1, N, 64), lambda i: (i, 0, 0)),
            pl.BlockSpec(w1.shape, lambda i: (0, 0, 0)),
            pl.BlockSpec(b1.shape, lambda i: (0, 0)),
            pl.BlockSpec(w2.shape, lambda i: (0, 0, 0)),
            pl.BlockSpec(b2.shape, lambda i: (0, 0)),
            pl.BlockSpec(w3.shape, lambda i: (0, 0, 0)),
            pl.BlockSpec(b3.shape, lambda i: (0, 0)),
        ],
        out_specs=pl.BlockSpec((1, 112, 112, 12), lambda i: (i, 0, 0, 0)),
        out_shape=jax.ShapeDtypeStruct((B, 112, 112, 12), F32),
        scratch_shapes=[pltpu.VMEM((13000, 128), F32)],
    )(q_flat, w1, b1, w2, b2, w3, b3)


def _enc23vq_body(x_ref, w2_ref, b2_ref, w3_ref, b3_ref, cbt_ref,
                  idx_ref, cmt_ref):
    M = 56 * 58
    xc = jnp.concatenate([x_ref[0, pl.ds(o, M), :] for o in (0, 1, 58, 59)],
                         axis=1)
    h = jnp.dot(xc, w2_ref[...], preferred_element_type=F32)
    h = jnp.maximum(h + b2_ref[...], 0.0)
    col = lax.rem(lax.broadcasted_iota(jnp.int32, (M, 1), 0), jnp.int32(58))
    h = jnp.where(col < 56, h, 0.0)
    vp = jnp.concatenate([jnp.zeros((59, 64), F32), h,
                          jnp.zeros((61, 64), F32)], axis=0)
    xc3 = jnp.concatenate(
        [vp[dy * 58 + dx:dy * 58 + dx + M, :]
         for dy in range(3) for dx in range(3)], axis=1)
    z = jnp.dot(xc3, w3_ref[...], preferred_element_type=F32) + b3_ref[...]

    cbt = cbt_ref[...]
    z2 = jnp.sum(z * z, axis=1, keepdims=True)
    dot = jnp.dot(z, cbt, preferred_element_type=F32)
    cb2 = jnp.sum(cbt * cbt, axis=0, keepdims=True)
    d2 = (z2 - 2.0 * dot) + cb2
    dmin = jnp.min(d2, axis=1, keepdims=True)
    lanes = lax.broadcasted_iota(jnp.int32, d2.shape, 1)
    idx = jnp.min(jnp.where(d2 == dmin, lanes, jnp.int32(d2.shape[1])),
                  axis=1)
    idx_ref[0, 0] = idx

    @pl.when(pl.program_id(0) == 0)
    def _():
        cmt_ref[...] = jnp.zeros_like(cmt_ref)

    cmt_ref[...] += jnp.sum(jnp.where(col < 56, dmin, 0.0))


def _enc23vq(x_flat, w2, b2, w3, b3, cbt):
    B, N, Cin = x_flat.shape
    idx, cmt = pl.pallas_call(
        _enc23vq_body,
        grid=(B,),
        in_specs=[
            pl.BlockSpec((1, N, Cin), lambda i: (i, 0, 0)),
            pl.BlockSpec(w2.shape, lambda i: (0, 0)),
            pl.BlockSpec(b2.shape, lambda i: (0, 0)),
            pl.BlockSpec(w3.shape, lambda i: (0, 0)),
            pl.BlockSpec(b3.shape, lambda i: (0, 0)),
            pl.BlockSpec(cbt.shape, lambda i: (0, 0)),
        ],
        out_specs=[
            pl.BlockSpec((1, 1, 56 * 58), lambda i: (i, 0, 0)),
            pl.BlockSpec((8, 128), lambda i: (0, 0)),
        ],
        out_shape=[
            jax.ShapeDtypeStruct((B, 1, 56 * 58), jnp.int32),
            jax.ShapeDtypeStruct((8, 128), F32),
        ],
    )(x_flat, w2, b2, w3, b3, cbt)
    return idx, cmt[0, 0]


def _vq_body(z_ref, cbt_ref, idx_ref, cmt_ref):
    z = z_ref[...]
    cbt = cbt_ref[...]
    z2 = jnp.sum(z * z, axis=1, keepdims=True)
    dot = jnp.dot(z, cbt, preferred_element_type=F32)
    cb2 = jnp.sum(cbt * cbt, axis=0, keepdims=True)
    d2 = (z2 - 2.0 * dot) + cb2
    dmin = jnp.min(d2, axis=1, keepdims=True)
    lanes = lax.broadcasted_iota(jnp.int32, d2.shape, 1)
    idx = jnp.min(jnp.where(d2 == dmin, lanes, jnp.int32(d2.shape[1])),
                  axis=1)
    idx_ref[0, 0] = idx

    @pl.when(pl.program_id(0) == 0)
    def _():
        cmt_ref[...] = jnp.zeros_like(cmt_ref)

    cmt_ref[...] += jnp.sum(dmin)


def _vq(zf, codebook, tile):
    R = zf.shape[0]
    K, D = codebook.shape
    grid = (R // tile,)
    idx, cmt = pl.pallas_call(
        _vq_body,
        grid=grid,
        in_specs=[
            pl.BlockSpec((tile, D), lambda i: (i, 0)),
            pl.BlockSpec((D, K), lambda i: (0, 0)),
        ],
        out_specs=[
            pl.BlockSpec((1, 1, tile), lambda i: (i, 0, 0)),
            pl.BlockSpec((8, 128), lambda i: (0, 0)),
        ],
        out_shape=[
            jax.ShapeDtypeStruct((R // tile, 1, tile), jnp.int32),
            jax.ShapeDtypeStruct((8, 128), F32),
        ],
    )(zf, codebook.T)
    return idx.reshape(R), cmt[0, 0]


def _sc_gather(codebook, idx, D):
    R = idx.shape[0]
    info = plsc.get_sparse_core_info()
    nw = info.num_cores * info.num_subcores
    bpw = R // nw
    K = codebook.shape[0]
    tab_flat = codebook.T.reshape(-1)
    mesh = plsc.VectorSubcoreMesh(core_axis_name="c", subcore_axis_name="s")

    @functools.partial(
        pl.kernel, mesh=mesh,
        out_type=jax.ShapeDtypeStruct((nw, D * bpw), F32),
        scratch_types=[
            pltpu.VMEM((bpw,), jnp.int32),
            pltpu.VMEM((D * K,), F32),
            pltpu.VMEM((D * bpw,), F32),
        ],
        compiler_params=pltpu.CompilerParams(needs_layout_passes=False),
    )
    def k(table_hbm, idx_hbm, out_hbm, idx_v, tab_v, qT_v):
        wid = lax.axis_index("s") * info.num_cores + lax.axis_index("c")
        base = wid * bpw
        pltpu.sync_copy(idx_hbm.at[pl.ds(base, bpw)], idx_v)
        pltpu.sync_copy(table_hbm, tab_v)

        def body(j, carry):
            iv = idx_v[pl.ds(j * 16, 16)]
            for c in range(D):
                qT_v[pl.ds(c * bpw + j * 16, 16)] = plsc.load_gather(
                    tab_v, [iv + jnp.int32(c * K)])
            return carry

        lax.fori_loop(0, bpw // 16, body, 0)
        pltpu.sync_copy(qT_v, out_hbm.at[wid])

    out = k(tab_flat, idx)
    return out.reshape(nw, D, bpw).transpose(0, 2, 1).reshape(R, D)


def kernel(x, w1, b1, w2, b2, w3, b3, codebook, dw1, db1, dw2, db2, dw3, db3):
    B = x.shape[0]

    xh = x.transpose(0, 2, 3, 1)
    xs = _s2d(_pad_hw(xh, 1, 3))
    xs = _flatten_rows(xs, 0)
    h1 = _tapconv(xs, _conv_w(w1, True), b1[None, :],
                  offs=(0, 1, 114, 115), M=112 * 114, Ho=112, Wp=114, Wo=112,
                  act="relu")

    hs = _flatten_rows(_s2d(_pad_hw(h1, 1, 3)), 0)
    idx58, cmt_sum = _enc23vq(hs, _conv_w(w2, True), b2[None, :],
                              _conv_w(w3, False), b3[None, :], codebook.T)
    D = codebook.shape[1]
    idx = idx58.reshape(B, 56, 58)[:, :, :56].reshape(-1)
    cmt = cmt_sum / jnp.float32(idx.shape[0] * D)
    q = _sc_gather(codebook, idx, D)
    qmap = q.reshape(B, 56, 56, D)

    qp = _flatten_rows(_pad_hw(qmap, 1, 1), 4)
    offs_d = tuple(
        tuple((ry + dy) * 58 + (rx + dx) for dy in (0, 1) for dx in (0, 1))
        for ry in (0, 1) for rx in (0, 1))
    offs_d2 = tuple(
        tuple((ry + dy) * 114 + (rx + dx) for dy in (0, 1) for dx in (0, 1))
        for ry in (0, 1) for rx in (0, 1))
    dec_ph = _dec12(qp, _deconv_phase_w(dw1), db1[None, :],
                    _deconv_phase_w(dw2), db2[None, :],
                    _phase_conv3_w(dw3), jnp.tile(db3, 4)[None, :],
                    offs_d, offs_d2)
    dec = dec_ph.reshape(B, 112, 112, 2, 2, 3)
    dec = dec.transpose(0, 5, 1, 3, 2, 4).reshape(B, 3, 224, 224)
    return dec, idx.reshape(B, 56, 56), cmt


def _phase_conv3_w(dw3):
    wf = jnp.flip(dw3, (2, 3))
    W9 = jnp.zeros((9, 128, 12), F32)
    for dm in (-1, 0, 1):
        for dn in (-1, 0, 1):
            for ry in (0, 1):
                for rx in (0, 1):
                    for sy in (0, 1):
                        for sx in (0, 1):
                            t = 2 * dm + ry + 1 - sy
                            v = 2 * dn + rx + 1 - sx
                            if 0 <= t <= 2 and 0 <= v <= 2:
                                pin, pout = ry * 2 + rx, sy * 2 + sx
                                W9 = W9.at[(dm + 1) * 3 + (dn + 1),
                                           pin * 32:(pin + 1) * 32,
                                           pout * 3:(pout + 1) * 3
                                           ].set(wf[:, :, t, v])
    return W9


def _dconv3_body(x_ref, w_ref, b_ref, o_ref):
    M = 112 * 114
    acc = jnp.zeros((M, 12), F32) + b_ref[...]
    for tap in range(9):
        dm, dn = tap // 3, tap % 3
        acc = acc + jnp.dot(x_ref[0, pl.ds(dm * 114 + dn, M), :], w_ref[tap],
                            preferred_element_type=F32)
    acc = jnp.tanh(acc)
    o_ref[0] = acc.reshape(112, 114, 12)[:, :112, :]


def _dconv3_phase(x_flat, w9, b):
    B, N, Cin = x_flat.shape
    return pl.pallas_call(
        _dconv3_body,
        grid=(B,),
        in_specs=[
            pl.BlockSpec((1, N, Cin), lambda i: (i, 0, 0)),
            pl.BlockSpec(w9.shape, lambda i: (0, 0, 0)),
            pl.BlockSpec(b.shape, lambda i: (0, 0)),
        ],
        out_specs=pl.BlockSpec((1, 112, 112, 12), lambda i: (i, 0, 0, 0)),
        out_shape=jax.ShapeDtypeStruct((B, 112, 112, 12), F32),
    )(x_flat, w9, b)


# speedup vs baseline: 1.1311x; 1.1311x over previous
import functools

import jax
import jax.numpy as jnp
from jax import lax
from jax.experimental import pallas as pl
from jax.experimental.pallas import tpu as pltpu
from jax.experimental.pallas import tpu_sc as plsc

F32 = jnp.float32


def _pad_hw(x, lo, hi):
    return jnp.pad(x, ((0, 0), (lo, hi), (lo, hi), (0, 0)))


def _s2d(x):
    B, H, W, C = x.shape
    x = x.reshape(B, H // 2, 2, W // 2, 2, C)
    x = x.transpose(0, 1, 3, 2, 4, 5)
    return x.reshape(B, H // 2, W // 2, 4 * C)


def _flatten_rows(x, extra):
    B, H, W, C = x.shape
    x = x.reshape(B, H * W, C)
    return jnp.pad(x, ((0, 0), (0, extra), (0, 0)))


def _conv_w(w, s2d):
    O, I, kh, kw = w.shape
    wt = w.transpose(2, 3, 1, 0)
    if s2d:
        wt = wt.reshape(2, 2, 2, 2, I, O)
        wt = wt.transpose(0, 2, 1, 3, 4, 5)
        return wt.reshape(kh * kw * I, O)
    return wt.reshape(kh * kw * I, O)


def _deconv_phase_w(dw):
    kidx = ((3, 1), (2, 0))
    phases = []
    for ry in (0, 1):
        for rx in (0, 1):
            blocks = [dw[:, :, kidx[ry][dy], kidx[rx][dx]]
                      for dy in (0, 1) for dx in (0, 1)]
            phases.append(jnp.concatenate(blocks, axis=0))
    return jnp.stack(phases)


def _tapconv_body(x_ref, w_ref, b_ref, o_ref, *, offs, M, Ho, Wp, Wo, act):
    xc = jnp.concatenate([x_ref[0, pl.ds(o, M), :] for o in offs], axis=1)
    acc = jnp.dot(xc, w_ref[...], preferred_element_type=F32)
    acc = acc + b_ref[...]
    if act == "relu":
        acc = jnp.maximum(acc, 0.0)
    C = acc.shape[-1]
    o_ref[0] = acc.reshape(Ho, Wp, C)[:, :Wo, :]


def _tapconv(x_flat, w, b, offs, M, Ho, Wp, Wo, act):
    B, N, Cin = x_flat.shape
    Cout = w.shape[-1]
    body = functools.partial(_tapconv_body, offs=offs, M=M, Ho=Ho, Wp=Wp,
                             Wo=Wo, act=act)
    return pl.pallas_call(
        body,
        grid=(B,),
        in_specs=[
            pl.BlockSpec((1, N, Cin), lambda i: (i, 0, 0)),
            pl.BlockSpec(w.shape, lambda i: (0, 0)),
            pl.BlockSpec(b.shape, lambda i: (0, 0)),
        ],
        out_specs=pl.BlockSpec((1, Ho, Wo, Cout), lambda i: (i, 0, 0, 0)),
        out_shape=jax.ShapeDtypeStruct((B, Ho, Wo, Cout), F32),
    )(x_flat, w, b)


def _deconv_body(x_ref, w_ref, b_ref, o_ref, *, offs4, M, Ho, Wp, Wo, act):
    accs = []
    for ph in range(4):
        xc = jnp.concatenate([x_ref[0, pl.ds(o, M), :] for o in offs4[ph]],
                             axis=1)
        acc = jnp.dot(xc, w_ref[ph], preferred_element_type=F32)
        acc = acc + b_ref[...]
        if act == "relu":
            acc = jnp.maximum(acc, 0.0)
        accs.append(acc)
    cat = jnp.concatenate(accs, axis=1)
    C4 = cat.shape[-1]
    o_ref[0] = cat.reshape(Ho, Wp, C4)[:, :Wo, :]


def _deconv(x_flat, wph, b, offs4, M, Ho, Wp, Wo, act):
    B, N, Cin = x_flat.shape
    Cout = wph.shape[-1]
    body = functools.partial(_deconv_body, offs4=offs4, M=M, Ho=Ho, Wp=Wp,
                             Wo=Wo, act=act)
    return pl.pallas_call(
        body,
        grid=(B,),
        in_specs=[
            pl.BlockSpec((1, N, Cin), lambda i: (i, 0, 0)),
            pl.BlockSpec(wph.shape, lambda i: (0, 0, 0)),
            pl.BlockSpec(b.shape, lambda i: (0, 0)),
        ],
        out_specs=pl.BlockSpec((1, Ho, Wo, 4 * Cout), lambda i: (i, 0, 0, 0)),
        out_shape=jax.ShapeDtypeStruct((B, Ho, Wo, 4 * Cout), F32),
    )(x_flat, wph, b)


def _d2s(x, C):
    B, Mh, Nw, _ = x.shape
    x = x.reshape(B, Mh, Nw, 2, 2, C).transpose(0, 1, 3, 2, 4, 5)
    return x.reshape(B, 2 * Mh, 2 * Nw, C)


def _dec12_body(q_ref, w1_ref, b1_ref, w2_ref, b2_ref, o_ref, *,
                offs1, offs2):
    M1 = 56 * 58
    accs = []
    for ph in range(4):
        xc = jnp.concatenate([q_ref[0, pl.ds(o, M1), :] for o in offs1[ph]],
                             axis=1)
        acc = jnp.dot(xc, w1_ref[ph], preferred_element_type=F32)
        accs.append(jnp.maximum(acc + b1_ref[...], 0.0))
    cat = jnp.concatenate(accs, axis=1)
    v = cat.reshape(56, 58, 2, 2, 64).transpose(0, 2, 1, 3, 4)
    v = v.reshape(112, 116, 64)
    v = jnp.concatenate([jnp.zeros((112, 1, 64), F32), v[:, :112, :],
                         jnp.zeros((112, 1, 64), F32)],
                        axis=1)
    zrow = jnp.zeros((1, 114, 64), F32)
    v = jnp.concatenate([zrow, v, zrow], axis=0)
    vf = jnp.concatenate([v.reshape(12996, 64), jnp.zeros((4, 64), F32)],
                         axis=0)

    M2 = 112 * 114
    accs2 = []
    for ph in range(4):
        xc = jnp.concatenate([vf[o:o + M2, :] for o in offs2[ph]], axis=1)
        acc = jnp.dot(xc, w2_ref[ph], preferred_element_type=F32)
        accs2.append(jnp.maximum(acc + b2_ref[...], 0.0))
    cat2 = jnp.concatenate(accs2, axis=1)
    col = lax.rem(lax.broadcasted_iota(jnp.int32, (M2, 1), 0),
                  jnp.int32(114))
    cat2 = jnp.where(col < 112, cat2, 0.0)
    o_ref[0, 0:115, :] = jnp.zeros((115, 128), F32)
    o_ref[0, pl.ds(115, M2), :] = cat2
    o_ref[0, 12883:13000, :] = jnp.zeros((117, 128), F32)


def _dec12(q_flat, w1, b1, w2, b2, offs1, offs2):
    B, N, _ = q_flat.shape
    body = functools.partial(_dec12_body, offs1=offs1, offs2=offs2)
    return pl.pallas_call(
        body,
        grid=(B,),
        in_specs=[
            pl.BlockSpec((1, N, 64), lambda i: (i, 0, 0)),
            pl.BlockSpec(w1.shape, lambda i: (0, 0, 0)),
            pl.BlockSpec(b1.shape, lambda i: (0, 0)),
            pl.BlockSpec(w2.shape, lambda i: (0, 0, 0)),
            pl.BlockSpec(b2.shape, lambda i: (0, 0)),
        ],
        out_specs=pl.BlockSpec((1, 13000, 128), lambda i: (i, 0, 0)),
        out_shape=jax.ShapeDtypeStruct((B, 13000, 128), F32),
    )(q_flat, w1, b1, w2, b2)


def _enc23vq_body(x_ref, w2_ref, b2_ref, w3_ref, b3_ref, cbt_ref,
                  idx_ref, cmt_ref):
    M = 56 * 58
    xc = jnp.concatenate([x_ref[0, pl.ds(o, M), :] for o in (0, 1, 58, 59)],
                         axis=1)
    h = jnp.dot(xc, w2_ref[...], preferred_element_type=F32)
    h = jnp.maximum(h + b2_ref[...], 0.0)
    col = lax.rem(lax.broadcasted_iota(jnp.int32, (M, 1), 0), jnp.int32(58))
    h = jnp.where(col < 56, h, 0.0)
    vp = jnp.concatenate([jnp.zeros((59, 64), F32), h,
                          jnp.zeros((61, 64), F32)], axis=0)
    xc3 = jnp.concatenate(
        [vp[dy * 58 + dx:dy * 58 + dx + M, :]
         for dy in range(3) for dx in range(3)], axis=1)
    z = jnp.dot(xc3, w3_ref[...], preferred_element_type=F32) + b3_ref[...]

    cbt = cbt_ref[...]
    z2 = jnp.sum(z * z, axis=1, keepdims=True)
    dot = jnp.dot(z, cbt, preferred_element_type=F32)
    cb2 = jnp.sum(cbt * cbt, axis=0, keepdims=True)
    d2 = (z2 - 2.0 * dot) + cb2
    dmin = jnp.min(d2, axis=1, keepdims=True)
    lanes = lax.broadcasted_iota(jnp.int32, d2.shape, 1)
    idx = jnp.min(jnp.where(d2 == dmin, lanes, jnp.int32(d2.shape[1])),
                  axis=1)
    idx_ref[0, 0] = idx

    @pl.when(pl.program_id(0) == 0)
    def _():
        cmt_ref[...] = jnp.zeros_like(cmt_ref)

    cmt_ref[...] += jnp.sum(jnp.where(col < 56, dmin, 0.0))


def _enc23vq(x_flat, w2, b2, w3, b3, cbt):
    B, N, Cin = x_flat.shape
    idx, cmt = pl.pallas_call(
        _enc23vq_body,
        grid=(B,),
        in_specs=[
            pl.BlockSpec((1, N, Cin), lambda i: (i, 0, 0)),
            pl.BlockSpec(w2.shape, lambda i: (0, 0)),
            pl.BlockSpec(b2.shape, lambda i: (0, 0)),
            pl.BlockSpec(w3.shape, lambda i: (0, 0)),
            pl.BlockSpec(b3.shape, lambda i: (0, 0)),
            pl.BlockSpec(cbt.shape, lambda i: (0, 0)),
        ],
        out_specs=[
            pl.BlockSpec((1, 1, 56 * 58), lambda i: (i, 0, 0)),
            pl.BlockSpec((8, 128), lambda i: (0, 0)),
        ],
        out_shape=[
            jax.ShapeDtypeStruct((B, 1, 56 * 58), jnp.int32),
            jax.ShapeDtypeStruct((8, 128), F32),
        ],
    )(x_flat, w2, b2, w3, b3, cbt)
    return idx, cmt[0, 0]


def _vq_body(z_ref, cbt_ref, idx_ref, cmt_ref):
    z = z_ref[...]
    cbt = cbt_ref[...]
    z2 = jnp.sum(z * z, axis=1, keepdims=True)
    dot = jnp.dot(z, cbt, preferred_element_type=F32)
    cb2 = jnp.sum(cbt * cbt, axis=0, keepdims=True)
    d2 = (z2 - 2.0 * dot) + cb2
    dmin = jnp.min(d2, axis=1, keepdims=True)
    lanes = lax.broadcasted_iota(jnp.int32, d2.shape, 1)
    idx = jnp.min(jnp.where(d2 == dmin, lanes, jnp.int32(d2.shape[1])),
                  axis=1)
    idx_ref[0, 0] = idx

    @pl.when(pl.program_id(0) == 0)
    def _():
        cmt_ref[...] = jnp.zeros_like(cmt_ref)

    cmt_ref[...] += jnp.sum(dmin)


def _vq(zf, codebook, tile):
    R = zf.shape[0]
    K, D = codebook.shape
    grid = (R // tile,)
    idx, cmt = pl.pallas_call(
        _vq_body,
        grid=grid,
        in_specs=[
            pl.BlockSpec((tile, D), lambda i: (i, 0)),
            pl.BlockSpec((D, K), lambda i: (0, 0)),
        ],
        out_specs=[
            pl.BlockSpec((1, 1, tile), lambda i: (i, 0, 0)),
            pl.BlockSpec((8, 128), lambda i: (0, 0)),
        ],
        out_shape=[
            jax.ShapeDtypeStruct((R // tile, 1, tile), jnp.int32),
            jax.ShapeDtypeStruct((8, 128), F32),
        ],
    )(zf, codebook.T)
    return idx.reshape(R), cmt[0, 0]


def _sc_gather(codebook, idx, D):
    R = idx.shape[0]
    info = plsc.get_sparse_core_info()
    nw = info.num_cores * info.num_subcores
    bpw = R // nw
    K = codebook.shape[0]
    tab_flat = codebook.T.reshape(-1)
    mesh = plsc.VectorSubcoreMesh(core_axis_name="c", subcore_axis_name="s")

    @functools.partial(
        pl.kernel, mesh=mesh,
        out_type=jax.ShapeDtypeStruct((nw, D * bpw), F32),
        scratch_types=[
            pltpu.VMEM((bpw,), jnp.int32),
            pltpu.VMEM((D * K,), F32),
            pltpu.VMEM((D * bpw,), F32),
        ],
        compiler_params=pltpu.CompilerParams(needs_layout_passes=False),
    )
    def k(table_hbm, idx_hbm, out_hbm, idx_v, tab_v, qT_v):
        wid = lax.axis_index("s") * info.num_cores + lax.axis_index("c")
        base = wid * bpw
        pltpu.sync_copy(idx_hbm.at[pl.ds(base, bpw)], idx_v)
        pltpu.sync_copy(table_hbm, tab_v)

        def body(j, carry):
            iv = idx_v[pl.ds(j * 16, 16)]
            for c in range(D):
                qT_v[pl.ds(c * bpw + j * 16, 16)] = plsc.load_gather(
                    tab_v, [iv + jnp.int32(c * K)])
            return carry

        lax.fori_loop(0, bpw // 16, body, 0)
        pltpu.sync_copy(qT_v, out_hbm.at[wid])

    out = k(tab_flat, idx)
    return out.reshape(nw, D, bpw).transpose(0, 2, 1).reshape(R, D)


def kernel(x, w1, b1, w2, b2, w3, b3, codebook, dw1, db1, dw2, db2, dw3, db3):
    B = x.shape[0]

    xh = x.transpose(0, 2, 3, 1)
    xs = _s2d(_pad_hw(xh, 1, 3))
    xs = _flatten_rows(xs, 0)
    h1 = _tapconv(xs, _conv_w(w1, True), b1[None, :],
                  offs=(0, 1, 114, 115), M=112 * 114, Ho=112, Wp=114, Wo=112,
                  act="relu")

    hs = _flatten_rows(_s2d(_pad_hw(h1, 1, 3)), 0)
    idx58, cmt_sum = _enc23vq(hs, _conv_w(w2, True), b2[None, :],
                              _conv_w(w3, False), b3[None, :], codebook.T)
    D = codebook.shape[1]
    idx = idx58.reshape(B, 56, 58)[:, :, :56].reshape(-1)
    cmt = cmt_sum / jnp.float32(idx.shape[0] * D)
    q = _sc_gather(codebook, idx, D)
    qmap = q.reshape(B, 56, 56, D)

    qp = _flatten_rows(_pad_hw(qmap, 1, 1), 4)
    offs_d = tuple(
        tuple((ry + dy) * 58 + (rx + dx) for dy in (0, 1) for dx in (0, 1))
        for ry in (0, 1) for rx in (0, 1))
    offs_d2 = tuple(
        tuple((ry + dy) * 114 + (rx + dx) for dy in (0, 1) for dx in (0, 1))
        for ry in (0, 1) for rx in (0, 1))
    g2p = _dec12(qp, _deconv_phase_w(dw1), db1[None, :],
                 _deconv_phase_w(dw2), db2[None, :],
                 offs_d, offs_d2)
    dec_ph = _dconv3_phase(g2p, _phase_conv3_w(dw3),
                           jnp.tile(db3, 4)[None, :])
    dec = dec_ph.reshape(B, 112, 112, 2, 2, 3)
    dec = dec.transpose(0, 5, 1, 3, 2, 4).reshape(B, 3, 224, 224)
    return dec, idx.reshape(B, 56, 56), cmt


def _phase_conv3_w(dw3):
    wf = jnp.flip(dw3, (2, 3))
    W9 = jnp.zeros((9, 128, 12), F32)
    for dm in (-1, 0, 1):
        for dn in (-1, 0, 1):
            for ry in (0, 1):
                for rx in (0, 1):
                    for sy in (0, 1):
                        for sx in (0, 1):
                            t = 2 * dm + ry + 1 - sy
                            v = 2 * dn + rx + 1 - sx
                            if 0 <= t <= 2 and 0 <= v <= 2:
                                pin, pout = ry * 2 + rx, sy * 2 + sx
                                W9 = W9.at[(dm + 1) * 3 + (dn + 1),
                                           pin * 32:(pin + 1) * 32,
                                           pout * 3:(pout + 1) * 3
                                           ].set(wf[:, :, t, v])
    return W9


def _dconv3_body(x_ref, w_ref, b_ref, o_ref):
    M = 112 * 114
    acc = jnp.zeros((M, 12), F32) + b_ref[...]
    for tap in range(9):
        dm, dn = tap // 3, tap % 3
        acc = acc + jnp.dot(x_ref[0, pl.ds(dm * 114 + dn, M), :], w_ref[tap],
                            preferred_element_type=F32)
    acc = jnp.tanh(acc)
    o_ref[0] = acc.reshape(112, 114, 12)[:, :112, :]


def _dconv3_phase(x_flat, w9, b):
    B, N, Cin = x_flat.shape
    return pl.pallas_call(
        _dconv3_body,
        grid=(B,),
        in_specs=[
            pl.BlockSpec((1, N, Cin), lambda i: (i, 0, 0)),
            pl.BlockSpec(w9.shape, lambda i: (0, 0, 0)),
            pl.BlockSpec(b.shape, lambda i: (0, 0)),
        ],
        out_specs=pl.BlockSpec((1, 112, 112, 12), lambda i: (i, 0, 0, 0)),
        out_shape=jax.ShapeDtypeStruct((B, 112, 112, 12), F32),
    )(x_flat, w9, b)
